# baseline (speedup 1.0000x reference)
import functools

import jax
import jax.numpy as jnp
from jax import lax
from jax.experimental import pallas as pl
from jax.experimental.pallas import tpu as pltpu
from jax.experimental.pallas import tpu_sc as plsc


@functools.cache
def _make_kernel(B, D):
    info = plsc.get_sparse_core_info()
    NC, NS = info.num_cores, info.num_subcores
    NW = NC * NS
    b_per_w = B // NW
    CH = 128
    n_ch = b_per_w // CH
    NSLOT = 3

    mesh = plsc.VectorSubcoreMesh(core_axis_name="c", subcore_axis_name="s")

    @functools.partial(
        pl.kernel,
        mesh=mesh,
        out_type=jax.ShapeDtypeStruct((B, D), jnp.float32),
        scratch_types=[
            pltpu.VMEM((n_ch, CH), jnp.int32),
            pltpu.VMEM((n_ch, CH), jnp.int32),
            pltpu.VMEM((NSLOT, CH, D), jnp.float32),
        ]
        + [pltpu.SemaphoreType.DMA] * n_ch
        + [pltpu.SemaphoreType.DMA] * NSLOT
        + [pltpu.SemaphoreType.DMA] * NSLOT,
    )
    def k(lf_hbm, seg_hbm, bidx_hbm, out_hbm, bidx_v, segid_v, rows_v,
          *sems):
        ssem = sems[:n_ch]
        gsem = sems[n_ch:n_ch + NSLOT]
        wsem = sems[n_ch + NSLOT:]
        wid = lax.axis_index("s") * NC + lax.axis_index("c")
        base = wid * b_per_w

        pltpu.sync_copy(bidx_hbm.at[wid], bidx_v)

        s = [
            pltpu.async_copy(seg_hbm.at[bidx_v.at[j]], segid_v.at[j], ssem[j])
            for j in range(n_ch)
        ]

        def fire_gather(j):
            return pltpu.async_copy(
                lf_hbm.at[segid_v.at[j]], rows_v.at[j % NSLOT], gsem[j % NSLOT]
            )

        def fire_write(j):
            return pltpu.async_copy(
                rows_v.at[j % NSLOT],
                out_hbm.at[pl.ds(base + j * CH, CH)],
                wsem[j % NSLOT],
            )

        g = [None] * n_ch
        w = [None] * n_ch
        for j in range(n_ch):
            s[j].wait()
            if j >= NSLOT:
                w[j - NSLOT].wait()
            g[j] = fire_gather(j)
            if j >= 1:
                g[j - 1].wait()
                w[j - 1] = fire_write(j - 1)
        g[n_ch - 1].wait()
        w[n_ch - 1] = fire_write(n_ch - 1)
        for j in range(max(0, n_ch - NSLOT), n_ch):
            w[j].wait()

    return k


def kernel(lf_list, lf_seg, batch_indices):
    info = plsc.get_sparse_core_info()
    NW = info.num_cores * info.num_subcores
    B, D = batch_indices.shape[0], lf_list.shape[1]
    seg_flat = lf_seg.reshape(-1)
    B_sc = (B // 4) * 3
    bidx3 = batch_indices[:B_sc].reshape(NW, B_sc // NW // 128, 128)
    sc_part = _make_kernel(B_sc, D)(lf_list, seg_flat, bidx3)
    tc_ids = jnp.take(seg_flat, batch_indices[B_sc:], axis=0)
    tc_part = jnp.take(lf_list, tc_ids, axis=0)
    return jnp.concatenate([sc_part, tc_part], axis=0)


# speedup vs baseline: 1.7244x; 1.7244x over previous
import functools

import jax
import jax.numpy as jnp
from jax import lax
from jax.experimental import pallas as pl
from jax.experimental.pallas import tpu as pltpu
from jax.experimental.pallas import tpu_sc as plsc


@functools.cache
def _make_kernel(B, D):
    info = plsc.get_sparse_core_info()
    NC, NS = info.num_cores, info.num_subcores
    NW = NC * NS
    b_per_w = B // NW
    CH = 128
    n_ch = b_per_w // CH
    NSLOT = 3

    mesh = plsc.VectorSubcoreMesh(core_axis_name="c", subcore_axis_name="s")

    @functools.partial(
        pl.kernel,
        mesh=mesh,
        out_type=jax.ShapeDtypeStruct((B, D), jnp.float32),
        scratch_types=[
            pltpu.VMEM((b_per_w,), jnp.int32),
            pltpu.VMEM((b_per_w,), jnp.int32),
            pltpu.VMEM((NSLOT, CH, D), jnp.float32),
        ]
        + [pltpu.SemaphoreType.DMA] * n_ch
        + [pltpu.SemaphoreType.DMA] * NSLOT
        + [pltpu.SemaphoreType.DMA] * NSLOT,
    )
    def k(lf_hbm, seg_hbm, bidx_hbm, out_hbm, bidx_v, segid_v, rows_v,
          *sems):
        ssem = sems[:n_ch]
        gsem = sems[n_ch:n_ch + NSLOT]
        wsem = sems[n_ch + NSLOT:]
        wid = lax.axis_index("s") * NC + lax.axis_index("c")
        base = wid * b_per_w

        pltpu.sync_copy(bidx_hbm.at[pl.ds(base, b_per_w)], bidx_v)

        s = [
            pltpu.async_copy(
                seg_hbm.at[bidx_v.at[pl.ds(j * CH, CH)]],
                segid_v.at[pl.ds(j * CH, CH)],
                ssem[j],
            )
            for j in range(n_ch)
        ]

        def fire_gather(j):
            return pltpu.async_copy(
                lf_hbm.at[segid_v.at[pl.ds(j * CH, CH)]],
                rows_v.at[j % NSLOT],
                gsem[j % NSLOT],
            )

        def fire_write(j):
            return pltpu.async_copy(
                rows_v.at[j % NSLOT],
                out_hbm.at[pl.ds(base + j * CH, CH)],
                wsem[j % NSLOT],
            )

        g = [None] * n_ch
        w = [None] * n_ch
        for j in range(n_ch):
            s[j].wait()
            if j >= NSLOT:
                w[j - NSLOT].wait()
            g[j] = fire_gather(j)
            if j >= 1:
                g[j - 1].wait()
                w[j - 1] = fire_write(j - 1)
        g[n_ch - 1].wait()
        w[n_ch - 1] = fire_write(n_ch - 1)
        for j in range(max(0, n_ch - NSLOT), n_ch):
            w[j].wait()

    return k


def kernel(lf_list, lf_seg, batch_indices):
    B, D = batch_indices.shape[0], lf_list.shape[1]
    seg_flat = lf_seg.reshape(-1)
    return _make_kernel(B, D)(lf_list, seg_flat, batch_indices)


# speedup vs baseline: 1.7325x; 1.0047x over previous
import functools

import jax
import jax.numpy as jnp
from jax import lax
from jax.experimental import pallas as pl
from jax.experimental.pallas import tpu as pltpu
from jax.experimental.pallas import tpu_sc as plsc


@functools.cache
def _make_kernel(B, D):
    info = plsc.get_sparse_core_info()
    NC, NS = info.num_cores, info.num_subcores
    NW = NC * NS
    b_per_w = B // NW
    CH = 128
    n_ch = b_per_w // CH
    NSLOT = 3

    mesh = plsc.VectorSubcoreMesh(core_axis_name="c", subcore_axis_name="s")

    @functools.partial(
        pl.kernel,
        mesh=mesh,
        out_type=jax.ShapeDtypeStruct((B, D), jnp.float32),
        scratch_types=[
            pltpu.VMEM((b_per_w,), jnp.int32),
            pltpu.VMEM((b_per_w,), jnp.int32),
            pltpu.VMEM((NSLOT, CH, D), jnp.float32),
        ]
        + [pltpu.SemaphoreType.DMA] * n_ch
        + [pltpu.SemaphoreType.DMA] * NSLOT
        + [pltpu.SemaphoreType.DMA] * NSLOT,
    )
    def k(lf_hbm, seg_hbm, bidx_hbm, out_hbm, bidx_v, segid_v, rows_v,
          *sems):
        ssem = sems[:n_ch]
        gsem = sems[n_ch:n_ch + NSLOT]
        wsem = sems[n_ch + NSLOT:]
        wid = lax.axis_index("c") * NS + lax.axis_index("s")
        base = wid * b_per_w

        pltpu.sync_copy(bidx_hbm.at[pl.ds(base, b_per_w)], bidx_v)

        s = [
            pltpu.async_copy(
                seg_hbm.at[bidx_v.at[pl.ds(j * CH, CH)]],
                segid_v.at[pl.ds(j * CH, CH)],
                ssem[j],
            )
            for j in range(n_ch)
        ]

        def fire_gather(j):
            return pltpu.async_copy(
                lf_hbm.at[segid_v.at[pl.ds(j * CH, CH)]],
                rows_v.at[j % NSLOT],
                gsem[j % NSLOT],
            )

        def fire_write(j):
            return pltpu.async_copy(
                rows_v.at[j % NSLOT],
                out_hbm.at[pl.ds(base + j * CH, CH)],
                wsem[j % NSLOT],
            )

        g = [None] * n_ch
        w = [None] * n_ch
        for j in range(n_ch):
            s[j].wait()
            if j >= NSLOT:
                w[j - NSLOT].wait()
            g[j] = fire_gather(j)
            if j >= 1:
                g[j - 1].wait()
                w[j - 1] = fire_write(j - 1)
        g[n_ch - 1].wait()
        w[n_ch - 1] = fire_write(n_ch - 1)
        for j in range(max(0, n_ch - NSLOT), n_ch):
            w[j].wait()

    return k


def kernel(lf_list, lf_seg, batch_indices):
    B, D = batch_indices.shape[0], lf_list.shape[1]
    seg_flat = lf_seg.reshape(-1)
    return _make_kernel(B, D)(lf_list, seg_flat, batch_indices)
